# Initial kernel scaffold; baseline (speedup 1.0000x reference)
import functools

import jax
import jax.numpy as jnp
from jax import lax
from jax.experimental import pallas as pl
from jax.experimental.pallas import tpu as pltpu
from jax.experimental.pallas import tpu_sc as plsc

N = 10000
E = 160000
NC = 2
NS = 16
ROWS_PER_TILE = N // NS
EDGES_PER_TILE = E // NS
EK = 400
NCHUNKS = EDGES_PER_TILE // EK
NPAD = 10240
EMB_ROWS = NPAD // (NC * NS)

_mesh = plsc.VectorSubcoreMesh(core_axis_name="c", subcore_axis_name="s")


def _sc_deg_emb_body(src_h, dst_h, gid_h, emb_h, zeros_h, ones_h,
                     deg_h, embg_h,
                     idx_v, ones_v, gidx_v, erows_v, acc_sh, sem):
    c = lax.axis_index("c")
    s = lax.axis_index("s")
    rb = pl.multiple_of(s * ROWS_PER_TILE, ROWS_PER_TILE)
    pltpu.sync_copy(zeros_h.at[pl.ds(rb, ROWS_PER_TILE)],
                    acc_sh.at[pl.ds(rb, ROWS_PER_TILE)])
    pltpu.sync_copy(ones_h, ones_v)
    plsc.subcore_barrier()

    def deg_loop(edge_h):
        def chunk(j, carry):
            eb = pl.multiple_of(s * EDGES_PER_TILE + j * EK, 8)
            pltpu.sync_copy(edge_h.at[pl.ds(eb, EK)], idx_v)
            pltpu.sync_copy(ones_v, acc_sh.at[idx_v], add=True)
            return carry
        lax.fori_loop(0, NCHUNKS, chunk, 0)

    @pl.when(c == 0)
    def _():
        deg_loop(src_h)

    @pl.when(c == 1)
    def _():
        deg_loop(dst_h)

    wid = s * NC + c
    gb = pl.multiple_of(wid * EMB_ROWS, 8)
    pltpu.sync_copy(gid_h.at[pl.ds(gb, EMB_ROWS)], gidx_v)
    pltpu.async_copy(emb_h.at[gidx_v], erows_v, sem).wait()
    pltpu.sync_copy(erows_v, embg_h.at[pl.ds(gb, EMB_ROWS)])

    plsc.subcore_barrier()

    @pl.when(c == 0)
    def _():
        pltpu.sync_copy(acc_sh.at[pl.ds(rb, ROWS_PER_TILE)],
                        deg_h.at[0].at[pl.ds(rb, ROWS_PER_TILE)])

    @pl.when(c == 1)
    def _():
        pltpu.sync_copy(acc_sh.at[pl.ds(rb, ROWS_PER_TILE)],
                        deg_h.at[1].at[pl.ds(rb, ROWS_PER_TILE)])


_sc_deg_emb = functools.partial(
    pl.kernel,
    out_type=(jax.ShapeDtypeStruct((2, N, 16), jnp.float32),
              jax.ShapeDtypeStruct((NPAD, 32), jnp.float32)),
    mesh=_mesh,
    scratch_types=[
        pltpu.VMEM((EK,), jnp.int32),
        pltpu.VMEM((EK, 16), jnp.float32),
        pltpu.VMEM((EMB_ROWS,), jnp.int32),
        pltpu.VMEM((EMB_ROWS, 32), jnp.float32),
        pltpu.VMEM_SHARED((N, 16), jnp.float32),
        pltpu.SemaphoreType.DMA,
    ],
)(_sc_deg_emb_body)


def _make_sc_scatter(fh):
    def body(y0_h, y1_h, src_h, dst_h, zeros_h, out_h,
             isrc_v, idst_v, rows_v, acc_sh, sem):
        c = lax.axis_index("c")
        s = lax.axis_index("s")
        rb = pl.multiple_of(s * ROWS_PER_TILE, ROWS_PER_TILE)
        pltpu.sync_copy(zeros_h.at[pl.ds(rb, ROWS_PER_TILE)],
                        acc_sh.at[pl.ds(rb, ROWS_PER_TILE)])
        plsc.subcore_barrier()

        def chunk(j, carry):
            eb = pl.multiple_of(s * EDGES_PER_TILE + j * EK, 8)
            pltpu.sync_copy(src_h.at[pl.ds(eb, EK)], isrc_v)
            pltpu.sync_copy(dst_h.at[pl.ds(eb, EK)], idst_v)

            @pl.when(c == 0)
            def _():
                pltpu.async_copy(y0_h.at[isrc_v], rows_v, sem).wait()

            @pl.when(c == 1)
            def _():
                pltpu.async_copy(y1_h.at[isrc_v], rows_v, sem).wait()

            pltpu.sync_copy(rows_v, acc_sh.at[idst_v], add=True)
            return carry

        lax.fori_loop(0, NCHUNKS, chunk, 0)
        plsc.subcore_barrier()

        @pl.when(c == 0)
        def _():
            pltpu.sync_copy(acc_sh.at[pl.ds(rb, ROWS_PER_TILE)],
                            out_h.at[0].at[pl.ds(rb, ROWS_PER_TILE)])

        @pl.when(c == 1)
        def _():
            pltpu.sync_copy(acc_sh.at[pl.ds(rb, ROWS_PER_TILE)],
                            out_h.at[1].at[pl.ds(rb, ROWS_PER_TILE)])

    return pl.kernel(
        body,
        out_type=jax.ShapeDtypeStruct((2, N, fh), jnp.float32),
        mesh=_mesh,
        scratch_types=[
            pltpu.VMEM((EK,), jnp.int32),
            pltpu.VMEM((EK,), jnp.int32),
            pltpu.VMEM((EK, fh), jnp.float32),
            pltpu.VMEM_SHARED((N, fh), jnp.float32),
            pltpu.SemaphoreType.DMA,
        ],
    )


_sc_scatter = {fh: _make_sc_scatter(fh) for fh in (128, 64, 32)}


_RB = 2000
_NBLK = N // _RB


def _norm(d):
    return lax.rsqrt(jnp.where(d > 0.0, d, 1.0))


def _tc_stage1_body(emb_ref, cbo_ref, enc_ref, wh_ref, bh_ref, dout_ref,
                    w1_ref, out_ref):
    t = jnp.dot(emb_ref[...], wh_ref[0:32, :],
                preferred_element_type=jnp.float32)
    t += jnp.dot(cbo_ref[...], wh_ref[32:128, :],
                 preferred_element_type=jnp.float32)
    t += jnp.dot(enc_ref[...], wh_ref[128:256, :],
                 preferred_element_type=jnp.float32)
    t = jnp.maximum(t + bh_ref[...], 0.0)
    x = t * _norm(dout_ref[...])
    y = jnp.dot(x, w1_ref[...], preferred_element_type=jnp.float32)
    out_ref[0] = y[:, :128]
    out_ref[1] = y[:, 128:]


def _tc_stage1(emb_g, cbo, enc, W_h, b_h, dout, W1):
    return pl.pallas_call(
        _tc_stage1_body,
        grid=(_NBLK,),
        in_specs=[
            pl.BlockSpec((_RB, 32), lambda i: (i, 0)),
            pl.BlockSpec((_RB, 96), lambda i: (i, 0)),
            pl.BlockSpec((_RB, 128), lambda i: (i, 0)),
            pl.BlockSpec((256, 512), lambda i: (0, 0)),
            pl.BlockSpec((1, 512), lambda i: (0, 0)),
            pl.BlockSpec((_RB, 1), lambda i: (i, 0)),
            pl.BlockSpec((512, 256), lambda i: (0, 0)),
        ],
        out_specs=pl.BlockSpec((2, _RB, 128), lambda i: (0, i, 0)),
        out_shape=jax.ShapeDtypeStruct((2, N, 128), jnp.float32),
    )(emb_g, cbo, enc, W_h, b_h, dout, W1)


def _make_tc_mid(fin, fout):
    fi2, fo2 = fin // 2, fout // 2

    def body(agg_ref, din_ref, dout_ref, b_ref, w_ref, out_ref):
        nd = _norm(din_ref[...])
        ns = _norm(dout_ref[...])
        x0 = jnp.maximum(agg_ref[0] * nd + b_ref[:, :fi2], 0.0) * ns
        x1 = jnp.maximum(agg_ref[1] * nd + b_ref[:, fi2:], 0.0) * ns
        y = jnp.dot(x0, w_ref[:fi2, :], preferred_element_type=jnp.float32)
        y += jnp.dot(x1, w_ref[fi2:, :], preferred_element_type=jnp.float32)
        out_ref[0] = y[:, :fo2]
        out_ref[1] = y[:, fo2:]

    def run(agg, din, dout, b, w):
        return pl.pallas_call(
            body,
            grid=(_NBLK,),
            in_specs=[
                pl.BlockSpec((2, _RB, fi2), lambda i: (0, i, 0)),
                pl.BlockSpec((_RB, 1), lambda i: (i, 0)),
                pl.BlockSpec((_RB, 1), lambda i: (i, 0)),
                pl.BlockSpec((1, fin), lambda i: (0, 0)),
                pl.BlockSpec((fin, fout), lambda i: (0, 0)),
            ],
            out_specs=pl.BlockSpec((2, _RB, fo2), lambda i: (0, i, 0)),
            out_shape=jax.ShapeDtypeStruct((2, N, fo2), jnp.float32),
        )(agg, din, dout, b, w)

    return run


_tc_mid = {(512, 256): _make_tc_mid(512, 256), (256, 128): _make_tc_mid(256, 128)}


def _tc_readout_body(agg_ref, din_ref, b3_ref, inst_ref, wm1_ref, bm1_ref,
                     wm2_ref, bm2_ref, wm3_ref, bm3_ref, out_ref, acc_ref):
    i = pl.program_id(0)

    @pl.when(i == 0)
    def _():
        acc_ref[...] = jnp.zeros_like(acc_ref)

    nd = _norm(din_ref[...])
    x0 = jnp.maximum(agg_ref[0] * nd + b3_ref[:, :32], 0.0)
    x1 = jnp.maximum(agg_ref[1] * nd + b3_ref[:, 32:], 0.0)
    acc_ref[:, :32] += jnp.sum(x0, axis=0, keepdims=True)
    acc_ref[:, 32:] += jnp.sum(x1, axis=0, keepdims=True)

    @pl.when(i == _NBLK - 1)
    def _():
        hg = acc_ref[...] * (1.0 / N)
        z = jnp.dot(hg, wm1_ref[:64, :], preferred_element_type=jnp.float32)
        z += jnp.dot(inst_ref[...], wm1_ref[64:, :],
                     preferred_element_type=jnp.float32)
        z = jnp.maximum(z + bm1_ref[...], 0.0)
        z = jnp.maximum(jnp.dot(z, wm2_ref[...],
                                preferred_element_type=jnp.float32)
                        + bm2_ref[...], 0.0)
        v = jnp.dot(z, wm3_ref[...], preferred_element_type=jnp.float32)
        out_ref[...] = jnp.exp(v + bm3_ref[...])


def _tc_readout(agg, din, b3, inst, Wm1, bm1, Wm2, bm2, Wm3, bm3):
    return pl.pallas_call(
        _tc_readout_body,
        grid=(_NBLK,),
        in_specs=[
            pl.BlockSpec((2, _RB, 32), lambda i: (0, i, 0)),
            pl.BlockSpec((_RB, 1), lambda i: (i, 0)),
            pl.BlockSpec((1, 64), lambda i: (0, 0)),
            pl.BlockSpec((1, 32), lambda i: (0, 0)),
            pl.BlockSpec((96, 256), lambda i: (0, 0)),
            pl.BlockSpec((1, 256), lambda i: (0, 0)),
            pl.BlockSpec((256, 256), lambda i: (0, 0)),
            pl.BlockSpec((1, 256), lambda i: (0, 0)),
            pl.BlockSpec((256, 1), lambda i: (0, 0)),
            pl.BlockSpec((1, 1), lambda i: (0, 0)),
        ],
        out_specs=pl.BlockSpec((1, 1), lambda i: (0, 0)),
        out_shape=jax.ShapeDtypeStruct((1, 1), jnp.float32),
        scratch_shapes=[pltpu.VMEM((1, 64), jnp.float32)],
    )(agg, din, b3, inst, Wm1, bm1, Wm2, bm2, Wm3, bm3)


def kernel(op_gid, cbo, enc, edge_index, inst_feat, emb_table, W_h, b_h,
           W1, b1, W2, b2, W3, b3, Wm1, bm1, Wm2, bm2, Wm3, bm3):
    src = edge_index[0]
    dst = edge_index[1]
    gid_pad = jnp.pad(op_gid.astype(jnp.int32), (0, NPAD - N))

    zeros16 = jnp.zeros((N, 16), jnp.float32)
    ones16 = jnp.ones((EK, 16), jnp.float32)
    deg, emb_pad = _sc_deg_emb(src, dst, gid_pad, emb_table, zeros16, ones16)
    emb_g = emb_pad[:N]
    dout = deg[0, :, 0:1]
    din = deg[1, :, 0:1]

    y1 = _tc_stage1(emb_g, cbo, enc, W_h, b_h.reshape(1, 512), dout, W1)
    agg1 = _sc_scatter[128](y1[0], y1[1], src, dst,
                            jnp.zeros((N, 128), jnp.float32))
    y2 = _tc_mid[(512, 256)](agg1, din, dout, b1.reshape(1, 256), W2)
    agg2 = _sc_scatter[64](y2[0], y2[1], src, dst,
                           jnp.zeros((N, 64), jnp.float32))
    y3 = _tc_mid[(256, 128)](agg2, din, dout, b2.reshape(1, 128), W3)
    agg3 = _sc_scatter[32](y3[0], y3[1], src, dst,
                           jnp.zeros((N, 32), jnp.float32))

    return _tc_readout(agg3, din, b3.reshape(1, 64), inst_feat,
                       Wm1, bm1.reshape(1, 256), Wm2, bm2.reshape(1, 256),
                       Wm3, bm3.reshape(1, 1))


# speedup vs baseline: 5.5345x; 5.5345x over previous
import functools

import jax
import jax.numpy as jnp
from jax import lax
from jax.experimental import pallas as pl
from jax.experimental.pallas import tpu as pltpu
from jax.experimental.pallas import tpu_sc as plsc

N = 10000
E = 160000
NC = 2
NS = 16
EDGES_PER_TILE = E // NS
EK = 400
NCHUNKS = EDGES_PER_TILE // EK
NPAD = 10240
ROWS_PER_TILE = NPAD // NS
EMB_ROWS = NPAD // (NC * NS)

_mesh = plsc.VectorSubcoreMesh(core_axis_name="c", subcore_axis_name="s")
_sc_params = pltpu.CompilerParams(use_tc_tiling_on_sc=False)


def _sc_deg_emb_body(src_h, dst_h, gid_h, emb_h, zeros_h, ones_h,
                     deg_h, embg_h,
                     idx_v, ones_v, gidx_v, erows_v, acc_sh, sem):
    c = lax.axis_index("c")
    s = lax.axis_index("s")
    rb = pl.multiple_of(s * ROWS_PER_TILE, ROWS_PER_TILE)
    pltpu.sync_copy(zeros_h.at[pl.ds(rb, ROWS_PER_TILE)],
                    acc_sh.at[pl.ds(rb, ROWS_PER_TILE)])
    pltpu.sync_copy(ones_h, ones_v)
    plsc.subcore_barrier()

    def deg_loop(edge_h):
        def chunk(j, carry):
            eb = pl.multiple_of(s * EDGES_PER_TILE + j * EK, 8)
            pltpu.sync_copy(edge_h.at[pl.ds(eb, EK)], idx_v)
            pltpu.sync_copy(ones_v, acc_sh.at[idx_v], add=True)
            return carry
        lax.fori_loop(0, NCHUNKS, chunk, 0)

    @pl.when(c == 0)
    def _():
        deg_loop(src_h)

    @pl.when(c == 1)
    def _():
        deg_loop(dst_h)

    wid = s * NC + c
    gb = pl.multiple_of(wid * EMB_ROWS, 8)
    pltpu.sync_copy(gid_h.at[pl.ds(gb, EMB_ROWS)], gidx_v)
    pltpu.async_copy(emb_h.at[gidx_v], erows_v, sem).wait()
    pltpu.sync_copy(erows_v, embg_h.at[pl.ds(gb, EMB_ROWS)])

    plsc.subcore_barrier()

    @pl.when(c == 0)
    def _():
        pltpu.sync_copy(acc_sh.at[pl.ds(rb, ROWS_PER_TILE)],
                        deg_h.at[0].at[pl.ds(rb, ROWS_PER_TILE)])

    @pl.when(c == 1)
    def _():
        pltpu.sync_copy(acc_sh.at[pl.ds(rb, ROWS_PER_TILE)],
                        deg_h.at[1].at[pl.ds(rb, ROWS_PER_TILE)])


_sc_deg_emb = functools.partial(
    pl.kernel,
    out_type=(jax.ShapeDtypeStruct((2, NPAD, 16), jnp.float32),
              jax.ShapeDtypeStruct((NPAD, 32), jnp.float32)),
    mesh=_mesh,
    scratch_types=[
        pltpu.VMEM((EK,), jnp.int32),
        pltpu.VMEM((EK, 16), jnp.float32),
        pltpu.VMEM((EMB_ROWS,), jnp.int32),
        pltpu.VMEM((EMB_ROWS, 32), jnp.float32),
        pltpu.VMEM_SHARED((NPAD, 16), jnp.float32),
        pltpu.SemaphoreType.DMA,
    ],
    compiler_params=_sc_params,
)(_sc_deg_emb_body)


def _make_sc_scatter(fh, dt):
    def body(y0_h, y1_h, src_h, dst_h, zeros_h, out_h,
             isrc_v, idst_v, rows_v, acc_sh, sem):
        c = lax.axis_index("c")
        s = lax.axis_index("s")
        rb = pl.multiple_of(s * ROWS_PER_TILE, ROWS_PER_TILE)
        pltpu.sync_copy(zeros_h.at[pl.ds(rb, ROWS_PER_TILE)],
                        acc_sh.at[pl.ds(rb, ROWS_PER_TILE)])
        plsc.subcore_barrier()

        def chunk(j, carry):
            eb = pl.multiple_of(s * EDGES_PER_TILE + j * EK, 8)
            pltpu.sync_copy(src_h.at[pl.ds(eb, EK)], isrc_v)
            pltpu.sync_copy(dst_h.at[pl.ds(eb, EK)], idst_v)

            @pl.when(c == 0)
            def _():
                pltpu.async_copy(y0_h.at[isrc_v], rows_v, sem).wait()

            @pl.when(c == 1)
            def _():
                pltpu.async_copy(y1_h.at[isrc_v], rows_v, sem).wait()

            pltpu.sync_copy(rows_v, acc_sh.at[idst_v], add=True)
            return carry

        lax.fori_loop(0, NCHUNKS, chunk, 0)
        plsc.subcore_barrier()

        @pl.when(c == 0)
        def _():
            pltpu.sync_copy(acc_sh.at[pl.ds(rb, ROWS_PER_TILE)],
                            out_h.at[0].at[pl.ds(rb, ROWS_PER_TILE)])

        @pl.when(c == 1)
        def _():
            pltpu.sync_copy(acc_sh.at[pl.ds(rb, ROWS_PER_TILE)],
                            out_h.at[1].at[pl.ds(rb, ROWS_PER_TILE)])

    return pl.kernel(
        body,
        out_type=jax.ShapeDtypeStruct((2, NPAD, fh), dt),
        mesh=_mesh,
        scratch_types=[
            pltpu.VMEM((EK,), jnp.int32),
            pltpu.VMEM((EK,), jnp.int32),
            pltpu.VMEM((EK, fh), dt),
            pltpu.VMEM_SHARED((NPAD, fh), dt),
            pltpu.SemaphoreType.DMA,
        ],
        compiler_params=_sc_params,
    )


_sc_scatter = {128: _make_sc_scatter(128, jnp.bfloat16),
               64: _make_sc_scatter(64, jnp.float32),
               32: _make_sc_scatter(32, jnp.float32)}


_RB = 2000
_NBLK = N // _RB


def _norm(d):
    return lax.rsqrt(jnp.where(d > 0.0, d, 1.0))


def _tc_stage1_body(emb_ref, cbo_ref, enc_ref, wh_ref, bh_ref, dout_ref,
                    w1_ref, out_ref):
    t = jnp.dot(emb_ref[...], wh_ref[0:32, :],
                preferred_element_type=jnp.float32)
    t += jnp.dot(cbo_ref[...], wh_ref[32:128, :],
                 preferred_element_type=jnp.float32)
    t += jnp.dot(enc_ref[...], wh_ref[128:256, :],
                 preferred_element_type=jnp.float32)
    t = jnp.maximum(t + bh_ref[...], 0.0)
    x = t * _norm(dout_ref[...])
    y = jnp.dot(x, w1_ref[...], preferred_element_type=jnp.float32)
    yb = y.astype(jnp.bfloat16)
    out_ref[0] = yb[:, :128]
    out_ref[1] = yb[:, 128:]


def _tc_stage1(emb_g, cbo, enc, W_h, b_h, dout, W1):
    return pl.pallas_call(
        _tc_stage1_body,
        grid=(_NBLK,),
        in_specs=[
            pl.BlockSpec((_RB, 32), lambda i: (i, 0)),
            pl.BlockSpec((_RB, 96), lambda i: (i, 0)),
            pl.BlockSpec((_RB, 128), lambda i: (i, 0)),
            pl.BlockSpec((256, 512), lambda i: (0, 0)),
            pl.BlockSpec((1, 512), lambda i: (0, 0)),
            pl.BlockSpec((_RB, 1), lambda i: (i, 0)),
            pl.BlockSpec((512, 256), lambda i: (0, 0)),
        ],
        out_specs=pl.BlockSpec((2, _RB, 128), lambda i: (0, i, 0)),
        out_shape=jax.ShapeDtypeStruct((2, N, 128), jnp.bfloat16),
    )(emb_g, cbo, enc, W_h, b_h, dout, W1)


def _make_tc_mid(fin, fout, in_dt):
    fi2, fo2 = fin // 2, fout // 2

    def body(agg_ref, din_ref, dout_ref, b_ref, w_ref, out_ref):
        nd = _norm(din_ref[...])
        ns = _norm(dout_ref[...])
        a0 = agg_ref[0].astype(jnp.float32)
        a1 = agg_ref[1].astype(jnp.float32)
        x0 = jnp.maximum(a0 * nd + b_ref[:, :fi2], 0.0) * ns
        x1 = jnp.maximum(a1 * nd + b_ref[:, fi2:], 0.0) * ns
        y = jnp.dot(x0, w_ref[:fi2, :], preferred_element_type=jnp.float32)
        y += jnp.dot(x1, w_ref[fi2:, :], preferred_element_type=jnp.float32)
        out_ref[0] = y[:, :fo2]
        out_ref[1] = y[:, fo2:]

    def run(agg, din, dout, b, w):
        return pl.pallas_call(
            body,
            grid=(_NBLK,),
            in_specs=[
                pl.BlockSpec((2, _RB, fi2), lambda i: (0, i, 0)),
                pl.BlockSpec((_RB, 1), lambda i: (i, 0)),
                pl.BlockSpec((_RB, 1), lambda i: (i, 0)),
                pl.BlockSpec((1, fin), lambda i: (0, 0)),
                pl.BlockSpec((fin, fout), lambda i: (0, 0)),
            ],
            out_specs=pl.BlockSpec((2, _RB, fo2), lambda i: (0, i, 0)),
            out_shape=jax.ShapeDtypeStruct((2, N, fo2), jnp.float32),
        )(agg, din, dout, b, w)

    return run


_tc_mid = {(256, 128): _make_tc_mid(256, 128, jnp.bfloat16),
           (128, 64): _make_tc_mid(128, 64, jnp.float32)}


def _tc_readout_body(agg_ref, din_ref, b3_ref, inst_ref, wm1_ref, bm1_ref,
                     wm2_ref, bm2_ref, wm3_ref, bm3_ref, out_ref, acc_ref):
    i = pl.program_id(0)

    @pl.when(i == 0)
    def _():
        acc_ref[...] = jnp.zeros_like(acc_ref)

    nd = _norm(din_ref[...])
    x0 = jnp.maximum(agg_ref[0] * nd + b3_ref[:, :32], 0.0)
    x1 = jnp.maximum(agg_ref[1] * nd + b3_ref[:, 32:], 0.0)
    acc_ref[:, :32] += jnp.sum(x0, axis=0, keepdims=True)
    acc_ref[:, 32:] += jnp.sum(x1, axis=0, keepdims=True)

    @pl.when(i == _NBLK - 1)
    def _():
        hg = acc_ref[...] * (1.0 / N)
        z = jnp.dot(hg, wm1_ref[:64, :], preferred_element_type=jnp.float32)
        z += jnp.dot(inst_ref[...], wm1_ref[64:, :],
                     preferred_element_type=jnp.float32)
        z = jnp.maximum(z + bm1_ref[...], 0.0)
        z = jnp.maximum(jnp.dot(z, wm2_ref[...],
                                preferred_element_type=jnp.float32)
                        + bm2_ref[...], 0.0)
        v = jnp.dot(z, wm3_ref[...], preferred_element_type=jnp.float32)
        out_ref[...] = jnp.exp(v + bm3_ref[...])


def _tc_readout(agg, din, b3, inst, Wm1, bm1, Wm2, bm2, Wm3, bm3):
    return pl.pallas_call(
        _tc_readout_body,
        grid=(_NBLK,),
        in_specs=[
            pl.BlockSpec((2, _RB, 32), lambda i: (0, i, 0)),
            pl.BlockSpec((_RB, 1), lambda i: (i, 0)),
            pl.BlockSpec((1, 64), lambda i: (0, 0)),
            pl.BlockSpec((1, 32), lambda i: (0, 0)),
            pl.BlockSpec((96, 256), lambda i: (0, 0)),
            pl.BlockSpec((1, 256), lambda i: (0, 0)),
            pl.BlockSpec((256, 256), lambda i: (0, 0)),
            pl.BlockSpec((1, 256), lambda i: (0, 0)),
            pl.BlockSpec((256, 1), lambda i: (0, 0)),
            pl.BlockSpec((1, 1), lambda i: (0, 0)),
        ],
        out_specs=pl.BlockSpec((1, 1), lambda i: (0, 0)),
        out_shape=jax.ShapeDtypeStruct((1, 1), jnp.float32),
        scratch_shapes=[pltpu.VMEM((1, 64), jnp.float32)],
    )(agg, din, b3, inst, Wm1, bm1, Wm2, bm2, Wm3, bm3)


def kernel(op_gid, cbo, enc, edge_index, inst_feat, emb_table, W_h, b_h,
           W1, b1, W2, b2, W3, b3, Wm1, bm1, Wm2, bm2, Wm3, bm3):
    src = edge_index[0]
    dst = edge_index[1]
    gid_pad = jnp.pad(op_gid.astype(jnp.int32), (0, NPAD - N))

    zeros16 = jnp.zeros((NPAD, 16), jnp.float32)
    ones16 = jnp.ones((EK, 16), jnp.float32)
    deg, emb_pad = _sc_deg_emb(src, dst, gid_pad, emb_table, zeros16, ones16)
    dout = deg[0, :N, 0:1]
    din = deg[1, :N, 0:1]

    y1 = _tc_stage1(emb_pad, cbo, enc, W_h, b_h.reshape(1, 512), dout, W1)
    agg1 = _sc_scatter[128](y1[0], y1[1], src, dst,
                            jnp.zeros((NPAD, 128), jnp.bfloat16))
    y2 = _tc_mid[(256, 128)](agg1, din, dout, b1.reshape(1, 256), W2)
    agg2 = _sc_scatter[64](y2[0], y2[1], src, dst,
                           jnp.zeros((NPAD, 64), jnp.float32))
    y3 = _tc_mid[(128, 64)](agg2, din, dout, b2.reshape(1, 128), W3)
    agg3 = _sc_scatter[32](y3[0], y3[1], src, dst,
                           jnp.zeros((NPAD, 32), jnp.float32))

    return _tc_readout(agg3, din, b3.reshape(1, 64), inst_feat,
                       Wm1, bm1.reshape(1, 256), Wm2, bm2.reshape(1, 256),
                       Wm3, bm3.reshape(1, 1))


# speedup vs baseline: 7.2704x; 1.3137x over previous
import functools

import jax
import jax.numpy as jnp
from jax import lax
from jax.experimental import pallas as pl
from jax.experimental.pallas import tpu as pltpu
from jax.experimental.pallas import tpu_sc as plsc

N = 10000
E = 160000
NC = 2
NS = 16
EDGES_PER_TILE = E // NS
EK = 400
NCHUNKS = EDGES_PER_TILE // EK
NPAD = 10240
ROWS_PER_TILE = NPAD // NS
EMB_ROWS = NPAD // (NC * NS)

_mesh = plsc.VectorSubcoreMesh(core_axis_name="c", subcore_axis_name="s")
_sc_params = pltpu.CompilerParams(use_tc_tiling_on_sc=False)


def _zero_rows(buf, nrows, fh, dt):
    lanes = 32 if dt == jnp.bfloat16 else 16
    zv = jnp.zeros((lanes,), dt)

    def zrow(r, carry):
        for q in range(fh // lanes):
            buf[r, pl.ds(q * lanes, lanes)] = zv
        return carry
    lax.fori_loop(0, nrows, zrow, 0)


def _zero_acc_slice(buf, acc_sh, rb):
    pltpu.sync_copy(buf, acc_sh.at[pl.ds(rb, EK)])
    pltpu.sync_copy(buf.at[pl.ds(0, ROWS_PER_TILE - EK)],
                    acc_sh.at[pl.ds(rb + EK, ROWS_PER_TILE - EK)])


def _sc_deg_emb_body(src_h, dst_h, gid_h, emb_h, deg_h, embg_h,
                     idx_v, ones_v, gidx_v, erows_v, acc_sh, sem):
    c = lax.axis_index("c")
    s = lax.axis_index("s")
    rb = pl.multiple_of(s * ROWS_PER_TILE, ROWS_PER_TILE)
    _zero_rows(ones_v, EK, 16, jnp.float32)
    _zero_acc_slice(ones_v, acc_sh, rb)
    onev = jnp.ones((16,), jnp.float32)

    def orow(r, carry):
        ones_v[r, :] = onev
        return carry
    lax.fori_loop(0, EK, orow, 0)
    plsc.subcore_barrier()

    def deg_loop(edge_h):
        def chunk(j, carry):
            eb = pl.multiple_of(s * EDGES_PER_TILE + j * EK, 8)
            pltpu.sync_copy(edge_h.at[pl.ds(eb, EK)], idx_v)
            pltpu.sync_copy(ones_v, acc_sh.at[idx_v], add=True)
            return carry
        lax.fori_loop(0, NCHUNKS, chunk, 0)

    @pl.when(c == 0)
    def _():
        deg_loop(src_h)

    @pl.when(c == 1)
    def _():
        deg_loop(dst_h)

    wid = s * NC + c
    gb = pl.multiple_of(wid * EMB_ROWS, 8)
    pltpu.sync_copy(gid_h.at[pl.ds(gb, EMB_ROWS)], gidx_v)
    pltpu.async_copy(emb_h.at[gidx_v], erows_v, sem).wait()
    pltpu.sync_copy(erows_v, embg_h.at[pl.ds(gb, EMB_ROWS)])

    plsc.subcore_barrier()

    @pl.when(c == 0)
    def _():
        pltpu.sync_copy(acc_sh.at[pl.ds(rb, ROWS_PER_TILE)],
                        deg_h.at[0].at[pl.ds(rb, ROWS_PER_TILE)])

    @pl.when(c == 1)
    def _():
        pltpu.sync_copy(acc_sh.at[pl.ds(rb, ROWS_PER_TILE)],
                        deg_h.at[1].at[pl.ds(rb, ROWS_PER_TILE)])


_sc_deg_emb = functools.partial(
    pl.kernel,
    out_type=(jax.ShapeDtypeStruct((2, NPAD, 16), jnp.float32),
              jax.ShapeDtypeStruct((NPAD, 32), jnp.float32)),
    mesh=_mesh,
    scratch_types=[
        pltpu.VMEM((EK,), jnp.int32),
        pltpu.VMEM((EK, 16), jnp.float32),
        pltpu.VMEM((EMB_ROWS,), jnp.int32),
        pltpu.VMEM((EMB_ROWS, 32), jnp.float32),
        pltpu.VMEM_SHARED((NPAD, 16), jnp.float32),
        pltpu.SemaphoreType.DMA,
    ],
    compiler_params=_sc_params,
)(_sc_deg_emb_body)


def _make_sc_scatter(fh, dt):
    def body(y_h, src_h, dst_h, out_h,
             is0, id0, is1, id1, r0, r1, acc_sh, sem0, sem1):
        c = lax.axis_index("c")
        s = lax.axis_index("s")
        rb = pl.multiple_of(s * ROWS_PER_TILE, ROWS_PER_TILE)
        _zero_rows(r0, EK, fh, dt)
        _zero_acc_slice(r0, acc_sh, rb)
        plsc.subcore_barrier()
        tb = s * EDGES_PER_TILE

        def stage(cix, isv, idv):
            eb = pl.multiple_of(tb + cix * EK, 8)
            pltpu.sync_copy(src_h.at[pl.ds(eb, EK)], isv)
            pltpu.sync_copy(dst_h.at[pl.ds(eb, EK)], idv)

        def gather(isv, rv, sem):
            @pl.when(c == 0)
            def _():
                pltpu.async_copy(y_h.at[0].at[isv], rv, sem)

            @pl.when(c == 1)
            def _():
                pltpu.async_copy(y_h.at[1].at[isv], rv, sem)

        def gwait(isv, rv, sem):
            @pl.when(c == 0)
            def _():
                pltpu.make_async_copy(y_h.at[0].at[isv], rv, sem).wait()

            @pl.when(c == 1)
            def _():
                pltpu.make_async_copy(y_h.at[1].at[isv], rv, sem).wait()

        stage(0, is0, id0)
        gather(is0, r0, sem0)

        def pair(k, carry):
            stage(2 * k + 1, is1, id1)
            gather(is1, r1, sem1)
            gwait(is0, r0, sem0)
            pltpu.sync_copy(r0, acc_sh.at[id0], add=True)
            stage(2 * k + 2, is0, id0)
            gather(is0, r0, sem0)
            gwait(is1, r1, sem1)
            pltpu.sync_copy(r1, acc_sh.at[id1], add=True)
            return carry

        lax.fori_loop(0, (NCHUNKS - 1) // 2, pair, 0)
        gwait(is0, r0, sem0)
        pltpu.sync_copy(r0, acc_sh.at[id0], add=True)
        plsc.subcore_barrier()

        @pl.when(c == 0)
        def _():
            pltpu.sync_copy(acc_sh.at[pl.ds(rb, ROWS_PER_TILE)],
                            out_h.at[0].at[pl.ds(rb, ROWS_PER_TILE)])

        @pl.when(c == 1)
        def _():
            pltpu.sync_copy(acc_sh.at[pl.ds(rb, ROWS_PER_TILE)],
                            out_h.at[1].at[pl.ds(rb, ROWS_PER_TILE)])

    return pl.kernel(
        body,
        out_type=jax.ShapeDtypeStruct((2, NPAD, fh), dt),
        mesh=_mesh,
        scratch_types=[
            pltpu.VMEM((EK,), jnp.int32),
            pltpu.VMEM((EK,), jnp.int32),
            pltpu.VMEM((EK,), jnp.int32),
            pltpu.VMEM((EK,), jnp.int32),
            pltpu.VMEM((EK, fh), dt),
            pltpu.VMEM((EK, fh), dt),
            pltpu.VMEM_SHARED((NPAD, fh), dt),
            pltpu.SemaphoreType.DMA,
            pltpu.SemaphoreType.DMA,
        ],
        compiler_params=_sc_params,
    )


_sc_scatter = {128: _make_sc_scatter(128, jnp.bfloat16),
               64: _make_sc_scatter(64, jnp.float32),
               32: _make_sc_scatter(32, jnp.float32)}


_RB = 2000
_NBLK = N // _RB


def _norm(d):
    return lax.rsqrt(jnp.where(d > 0.0, d, 1.0))


def _tc_stage1_body(emb_ref, cbo_ref, enc_ref, wh_ref, bh_ref, dout_ref,
                    w1_ref, out_ref):
    t = jnp.dot(emb_ref[...], wh_ref[0:32, :],
                preferred_element_type=jnp.float32)
    t += jnp.dot(cbo_ref[...], wh_ref[32:128, :],
                 preferred_element_type=jnp.float32)
    t += jnp.dot(enc_ref[...], wh_ref[128:256, :],
                 preferred_element_type=jnp.float32)
    t = jnp.maximum(t + bh_ref[...], 0.0)
    x = t * _norm(dout_ref[...])
    y = jnp.dot(x, w1_ref[...], preferred_element_type=jnp.float32)
    yb = y.astype(jnp.bfloat16)
    out_ref[0] = yb[:, :128]
    out_ref[1] = yb[:, 128:]


def _tc_stage1(emb_g, cbo, enc, W_h, b_h, dout, W1):
    return pl.pallas_call(
        _tc_stage1_body,
        grid=(_NBLK,),
        in_specs=[
            pl.BlockSpec((_RB, 32), lambda i: (i, 0)),
            pl.BlockSpec((_RB, 96), lambda i: (i, 0)),
            pl.BlockSpec((_RB, 128), lambda i: (i, 0)),
            pl.BlockSpec((256, 512), lambda i: (0, 0)),
            pl.BlockSpec((1, 512), lambda i: (0, 0)),
            pl.BlockSpec((_RB, 1), lambda i: (i, 0)),
            pl.BlockSpec((512, 256), lambda i: (0, 0)),
        ],
        out_specs=pl.BlockSpec((2, _RB, 128), lambda i: (0, i, 0)),
        out_shape=jax.ShapeDtypeStruct((2, N, 128), jnp.bfloat16),
    )(emb_g, cbo, enc, W_h, b_h, dout, W1)


def _make_tc_mid(fin, fout, in_dt):
    fi2, fo2 = fin // 2, fout // 2

    def body(agg_ref, din_ref, dout_ref, b_ref, w_ref, out_ref):
        nd = _norm(din_ref[...])
        ns = _norm(dout_ref[...])
        a0 = agg_ref[0].astype(jnp.float32)
        a1 = agg_ref[1].astype(jnp.float32)
        x0 = jnp.maximum(a0 * nd + b_ref[:, :fi2], 0.0) * ns
        x1 = jnp.maximum(a1 * nd + b_ref[:, fi2:], 0.0) * ns
        y = jnp.dot(x0, w_ref[:fi2, :], preferred_element_type=jnp.float32)
        y += jnp.dot(x1, w_ref[fi2:, :], preferred_element_type=jnp.float32)
        out_ref[0] = y[:, :fo2]
        out_ref[1] = y[:, fo2:]

    def run(agg, din, dout, b, w):
        return pl.pallas_call(
            body,
            grid=(_NBLK,),
            in_specs=[
                pl.BlockSpec((2, _RB, fi2), lambda i: (0, i, 0)),
                pl.BlockSpec((_RB, 1), lambda i: (i, 0)),
                pl.BlockSpec((_RB, 1), lambda i: (i, 0)),
                pl.BlockSpec((1, fin), lambda i: (0, 0)),
                pl.BlockSpec((fin, fout), lambda i: (0, 0)),
            ],
            out_specs=pl.BlockSpec((2, _RB, fo2), lambda i: (0, i, 0)),
            out_shape=jax.ShapeDtypeStruct((2, N, fo2), jnp.float32),
        )(agg, din, dout, b, w)

    return run


_tc_mid = {(256, 128): _make_tc_mid(256, 128, jnp.bfloat16),
           (128, 64): _make_tc_mid(128, 64, jnp.float32)}


def _tc_readout_body(agg_ref, din_ref, b3_ref, inst_ref, wm1_ref, bm1_ref,
                     wm2_ref, bm2_ref, wm3_ref, bm3_ref, out_ref, acc_ref):
    i = pl.program_id(0)

    @pl.when(i == 0)
    def _():
        acc_ref[...] = jnp.zeros_like(acc_ref)

    nd = _norm(din_ref[...])
    x0 = jnp.maximum(agg_ref[0] * nd + b3_ref[:, :32], 0.0)
    x1 = jnp.maximum(agg_ref[1] * nd + b3_ref[:, 32:], 0.0)
    acc_ref[:, :32] += jnp.sum(x0, axis=0, keepdims=True)
    acc_ref[:, 32:] += jnp.sum(x1, axis=0, keepdims=True)

    @pl.when(i == _NBLK - 1)
    def _():
        hg = acc_ref[...] * (1.0 / N)
        z = jnp.dot(hg, wm1_ref[:64, :], preferred_element_type=jnp.float32)
        z += jnp.dot(inst_ref[...], wm1_ref[64:, :],
                     preferred_element_type=jnp.float32)
        z = jnp.maximum(z + bm1_ref[...], 0.0)
        z = jnp.maximum(jnp.dot(z, wm2_ref[...],
                                preferred_element_type=jnp.float32)
                        + bm2_ref[...], 0.0)
        v = jnp.dot(z, wm3_ref[...], preferred_element_type=jnp.float32)
        out_ref[...] = jnp.exp(v + bm3_ref[...])


def _tc_readout(agg, din, b3, inst, Wm1, bm1, Wm2, bm2, Wm3, bm3):
    return pl.pallas_call(
        _tc_readout_body,
        grid=(_NBLK,),
        in_specs=[
            pl.BlockSpec((2, _RB, 32), lambda i: (0, i, 0)),
            pl.BlockSpec((_RB, 1), lambda i: (i, 0)),
            pl.BlockSpec((1, 64), lambda i: (0, 0)),
            pl.BlockSpec((1, 32), lambda i: (0, 0)),
            pl.BlockSpec((96, 256), lambda i: (0, 0)),
            pl.BlockSpec((1, 256), lambda i: (0, 0)),
            pl.BlockSpec((256, 256), lambda i: (0, 0)),
            pl.BlockSpec((1, 256), lambda i: (0, 0)),
            pl.BlockSpec((256, 1), lambda i: (0, 0)),
            pl.BlockSpec((1, 1), lambda i: (0, 0)),
        ],
        out_specs=pl.BlockSpec((1, 1), lambda i: (0, 0)),
        out_shape=jax.ShapeDtypeStruct((1, 1), jnp.float32),
        scratch_shapes=[pltpu.VMEM((1, 64), jnp.float32)],
    )(agg, din, b3, inst, Wm1, bm1, Wm2, bm2, Wm3, bm3)


def kernel(op_gid, cbo, enc, edge_index, inst_feat, emb_table, W_h, b_h,
           W1, b1, W2, b2, W3, b3, Wm1, bm1, Wm2, bm2, Wm3, bm3):
    src = edge_index[0]
    dst = edge_index[1]
    gid_pad = jnp.pad(op_gid.astype(jnp.int32), (0, NPAD - N))

    deg, emb_pad = _sc_deg_emb(src, dst, gid_pad, emb_table)
    dout = deg[0, :N, 0:1]
    din = deg[1, :N, 0:1]

    y1 = _tc_stage1(emb_pad, cbo, enc, W_h, b_h.reshape(1, 512), dout, W1)
    agg1 = _sc_scatter[128](y1, src, dst)
    y2 = _tc_mid[(256, 128)](agg1, din, dout, b1.reshape(1, 256), W2)
    agg2 = _sc_scatter[64](y2, src, dst)
    y3 = _tc_mid[(128, 64)](agg2, din, dout, b2.reshape(1, 128), W3)
    agg3 = _sc_scatter[32](y3, src, dst)

    return _tc_readout(agg3, din, b3.reshape(1, 64), inst_feat,
                       Wm1, bm1.reshape(1, 256), Wm2, bm2.reshape(1, 256),
                       Wm3, bm3.reshape(1, 1))


# speedup vs baseline: 7.7480x; 1.0657x over previous
import functools

import jax
import jax.numpy as jnp
from jax import lax
from jax.experimental import pallas as pl
from jax.experimental.pallas import tpu as pltpu
from jax.experimental.pallas import tpu_sc as plsc

N = 10000
E = 160000
NC = 2
NS = 16
EDGES_PER_TILE = E // NS
EK = 400
NCHUNKS = EDGES_PER_TILE // EK
NPAD = 10240
ROWS_PER_TILE = NPAD // NS
EMB_ROWS = NPAD // (NC * NS)

_mesh = plsc.VectorSubcoreMesh(core_axis_name="c", subcore_axis_name="s")
_sc_params = pltpu.CompilerParams(use_tc_tiling_on_sc=False)


def _zero_rows(buf, nrows, fh, dt):
    lanes = 32 if dt == jnp.bfloat16 else 16
    zv = jnp.zeros((lanes,), dt)

    def zrow(r, carry):
        for q in range(fh // lanes):
            buf[r, pl.ds(q * lanes, lanes)] = zv
        return carry
    lax.fori_loop(0, nrows, zrow, 0)


def _zero_acc_slice(buf, acc_sh, rb):
    pltpu.sync_copy(buf, acc_sh.at[pl.ds(rb, EK)])
    pltpu.sync_copy(buf.at[pl.ds(0, ROWS_PER_TILE - EK)],
                    acc_sh.at[pl.ds(rb + EK, ROWS_PER_TILE - EK)])


def _sc_deg_emb_body(src_h, dst_h, gid_h, emb_h, deg_h, embg_h,
                     idx_v, ones_v, gidx_v, erows_v, acc_sh, sem):
    c = lax.axis_index("c")
    s = lax.axis_index("s")
    rb = pl.multiple_of(s * ROWS_PER_TILE, ROWS_PER_TILE)
    _zero_rows(ones_v, EK, 16, jnp.float32)
    _zero_acc_slice(ones_v, acc_sh, rb)
    onev = jnp.ones((16,), jnp.float32)

    def orow(r, carry):
        ones_v[r, :] = onev
        return carry
    lax.fori_loop(0, EK, orow, 0)
    plsc.subcore_barrier()

    def deg_loop(edge_h):
        def chunk(j, carry):
            eb = pl.multiple_of(s * EDGES_PER_TILE + j * EK, 8)
            pltpu.sync_copy(edge_h.at[pl.ds(eb, EK)], idx_v)
            pltpu.sync_copy(ones_v, acc_sh.at[idx_v], add=True)
            return carry
        lax.fori_loop(0, NCHUNKS, chunk, 0)

    @pl.when(c == 0)
    def _():
        deg_loop(src_h)

    @pl.when(c == 1)
    def _():
        deg_loop(dst_h)

    wid = s * NC + c
    gb = pl.multiple_of(wid * EMB_ROWS, 8)
    pltpu.sync_copy(gid_h.at[pl.ds(gb, EMB_ROWS)], gidx_v)
    pltpu.async_copy(emb_h.at[gidx_v], erows_v, sem).wait()
    pltpu.sync_copy(erows_v, embg_h.at[pl.ds(gb, EMB_ROWS)])

    plsc.subcore_barrier()

    @pl.when(c == 0)
    def _():
        pltpu.sync_copy(acc_sh.at[pl.ds(rb, ROWS_PER_TILE)],
                        deg_h.at[0].at[pl.ds(rb, ROWS_PER_TILE)])

    @pl.when(c == 1)
    def _():
        pltpu.sync_copy(acc_sh.at[pl.ds(rb, ROWS_PER_TILE)],
                        deg_h.at[1].at[pl.ds(rb, ROWS_PER_TILE)])


_sc_deg_emb = functools.partial(
    pl.kernel,
    out_type=(jax.ShapeDtypeStruct((2, NPAD, 16), jnp.float32),
              jax.ShapeDtypeStruct((NPAD, 32), jnp.float32)),
    mesh=_mesh,
    scratch_types=[
        pltpu.VMEM((EK,), jnp.int32),
        pltpu.VMEM((EK, 16), jnp.float32),
        pltpu.VMEM((EMB_ROWS,), jnp.int32),
        pltpu.VMEM((EMB_ROWS, 32), jnp.float32),
        pltpu.VMEM_SHARED((NPAD, 16), jnp.float32),
        pltpu.SemaphoreType.DMA,
    ],
    compiler_params=_sc_params,
)(_sc_deg_emb_body)


def _make_sc_scatter(fh, dt):
    def body(y_h, src_h, dst_h, out_h,
             is0, id0, is1, id1, r0, r1, acc_sh, sem0, sem1):
        c = lax.axis_index("c")
        s = lax.axis_index("s")
        rb = pl.multiple_of(s * ROWS_PER_TILE, ROWS_PER_TILE)
        _zero_rows(r0, EK, fh, dt)
        _zero_acc_slice(r0, acc_sh, rb)
        plsc.subcore_barrier()
        tb = s * EDGES_PER_TILE

        def stage(cix, isv, idv):
            eb = pl.multiple_of(tb + cix * EK, 8)
            pltpu.sync_copy(src_h.at[pl.ds(eb, EK)], isv)
            pltpu.sync_copy(dst_h.at[pl.ds(eb, EK)], idv)

        def gather(isv, rv, sem):
            @pl.when(c == 0)
            def _():
                pltpu.async_copy(y_h.at[0].at[isv], rv, sem)

            @pl.when(c == 1)
            def _():
                pltpu.async_copy(y_h.at[1].at[isv], rv, sem)

        def gwait(isv, rv, sem):
            @pl.when(c == 0)
            def _():
                pltpu.make_async_copy(y_h.at[0].at[isv], rv, sem).wait()

            @pl.when(c == 1)
            def _():
                pltpu.make_async_copy(y_h.at[1].at[isv], rv, sem).wait()

        stage(0, is0, id0)
        gather(is0, r0, sem0)

        def pair(k, carry):
            stage(2 * k + 1, is1, id1)
            gather(is1, r1, sem1)
            gwait(is0, r0, sem0)
            pltpu.sync_copy(r0, acc_sh.at[id0], add=True)
            stage(2 * k + 2, is0, id0)
            gather(is0, r0, sem0)
            gwait(is1, r1, sem1)
            pltpu.sync_copy(r1, acc_sh.at[id1], add=True)
            return carry

        lax.fori_loop(0, (NCHUNKS - 1) // 2, pair, 0)
        gwait(is0, r0, sem0)
        pltpu.sync_copy(r0, acc_sh.at[id0], add=True)
        plsc.subcore_barrier()

        @pl.when(c == 0)
        def _():
            pltpu.sync_copy(acc_sh.at[pl.ds(rb, ROWS_PER_TILE)],
                            out_h.at[0].at[pl.ds(rb, ROWS_PER_TILE)])

        @pl.when(c == 1)
        def _():
            pltpu.sync_copy(acc_sh.at[pl.ds(rb, ROWS_PER_TILE)],
                            out_h.at[1].at[pl.ds(rb, ROWS_PER_TILE)])

    return pl.kernel(
        body,
        out_type=jax.ShapeDtypeStruct((2, NPAD, fh), dt),
        mesh=_mesh,
        scratch_types=[
            pltpu.VMEM((EK,), jnp.int32),
            pltpu.VMEM((EK,), jnp.int32),
            pltpu.VMEM((EK,), jnp.int32),
            pltpu.VMEM((EK,), jnp.int32),
            pltpu.VMEM((EK, fh), dt),
            pltpu.VMEM((EK, fh), dt),
            pltpu.VMEM_SHARED((NPAD, fh), dt),
            pltpu.SemaphoreType.DMA,
            pltpu.SemaphoreType.DMA,
        ],
        compiler_params=_sc_params,
    )


_sc_scatter = {128: _make_sc_scatter(128, jnp.bfloat16),
               64: _make_sc_scatter(64, jnp.bfloat16),
               32: _make_sc_scatter(32, jnp.bfloat16)}


_RB = 2000
_NBLK = N // _RB


def _norm(d):
    return lax.rsqrt(jnp.where(d > 0.0, d, 1.0))


def _tc_stage1_body(emb_ref, cbo_ref, enc_ref, wh_ref, bh_ref, dout_ref,
                    w1_ref, out_ref):
    t = jnp.dot(emb_ref[...], wh_ref[0:32, :],
                preferred_element_type=jnp.float32)
    t += jnp.dot(cbo_ref[...], wh_ref[32:128, :],
                 preferred_element_type=jnp.float32)
    t += jnp.dot(enc_ref[...], wh_ref[128:256, :],
                 preferred_element_type=jnp.float32)
    t = jnp.maximum(t + bh_ref[...], 0.0)
    x = t * _norm(dout_ref[...])
    y = jnp.dot(x, w1_ref[...], preferred_element_type=jnp.float32)
    yb = y.astype(jnp.bfloat16)
    out_ref[0] = yb[:, :128]
    out_ref[1] = yb[:, 128:]


def _tc_stage1(emb_g, cbo, enc, W_h, b_h, dout, W1):
    return pl.pallas_call(
        _tc_stage1_body,
        grid=(_NBLK,),
        in_specs=[
            pl.BlockSpec((_RB, 32), lambda i: (i, 0)),
            pl.BlockSpec((_RB, 96), lambda i: (i, 0)),
            pl.BlockSpec((_RB, 128), lambda i: (i, 0)),
            pl.BlockSpec((256, 512), lambda i: (0, 0)),
            pl.BlockSpec((1, 512), lambda i: (0, 0)),
            pl.BlockSpec((_RB, 1), lambda i: (i, 0)),
            pl.BlockSpec((512, 256), lambda i: (0, 0)),
        ],
        out_specs=pl.BlockSpec((2, _RB, 128), lambda i: (0, i, 0)),
        out_shape=jax.ShapeDtypeStruct((2, N, 128), jnp.bfloat16),
    )(emb_g, cbo, enc, W_h, b_h, dout, W1)


def _make_tc_mid(fin, fout, in_dt):
    fi2, fo2 = fin // 2, fout // 2

    def body(agg_ref, din_ref, dout_ref, b_ref, w_ref, out_ref):
        nd = _norm(din_ref[...])
        ns = _norm(dout_ref[...])
        a0 = agg_ref[0].astype(jnp.float32)
        a1 = agg_ref[1].astype(jnp.float32)
        x0 = jnp.maximum(a0 * nd + b_ref[:, :fi2], 0.0) * ns
        x1 = jnp.maximum(a1 * nd + b_ref[:, fi2:], 0.0) * ns
        y = jnp.dot(x0, w_ref[:fi2, :], preferred_element_type=jnp.float32)
        y += jnp.dot(x1, w_ref[fi2:, :], preferred_element_type=jnp.float32)
        yb = y.astype(jnp.bfloat16)
        out_ref[0] = yb[:, :fo2]
        out_ref[1] = yb[:, fo2:]

    def run(agg, din, dout, b, w):
        return pl.pallas_call(
            body,
            grid=(_NBLK,),
            in_specs=[
                pl.BlockSpec((2, _RB, fi2), lambda i: (0, i, 0)),
                pl.BlockSpec((_RB, 1), lambda i: (i, 0)),
                pl.BlockSpec((_RB, 1), lambda i: (i, 0)),
                pl.BlockSpec((1, fin), lambda i: (0, 0)),
                pl.BlockSpec((fin, fout), lambda i: (0, 0)),
            ],
            out_specs=pl.BlockSpec((2, _RB, fo2), lambda i: (0, i, 0)),
            out_shape=jax.ShapeDtypeStruct((2, N, fo2), jnp.bfloat16),
        )(agg, din, dout, b, w)

    return run


_tc_mid = {(256, 128): _make_tc_mid(256, 128, jnp.bfloat16),
           (128, 64): _make_tc_mid(128, 64, jnp.float32)}


def _tc_readout_body(agg_ref, din_ref, b3_ref, inst_ref, wm1_ref, bm1_ref,
                     wm2_ref, bm2_ref, wm3_ref, bm3_ref, out_ref, acc_ref):
    i = pl.program_id(0)

    @pl.when(i == 0)
    def _():
        acc_ref[...] = jnp.zeros_like(acc_ref)

    nd = _norm(din_ref[...])
    x0 = jnp.maximum(agg_ref[0].astype(jnp.float32) * nd + b3_ref[:, :32], 0.0)
    x1 = jnp.maximum(agg_ref[1].astype(jnp.float32) * nd + b3_ref[:, 32:], 0.0)
    acc_ref[:, :32] += jnp.sum(x0, axis=0, keepdims=True)
    acc_ref[:, 32:] += jnp.sum(x1, axis=0, keepdims=True)

    @pl.when(i == _NBLK - 1)
    def _():
        hg = acc_ref[...] * (1.0 / N)
        z = jnp.dot(hg, wm1_ref[:64, :], preferred_element_type=jnp.float32)
        z += jnp.dot(inst_ref[...], wm1_ref[64:, :],
                     preferred_element_type=jnp.float32)
        z = jnp.maximum(z + bm1_ref[...], 0.0)
        z = jnp.maximum(jnp.dot(z, wm2_ref[...],
                                preferred_element_type=jnp.float32)
                        + bm2_ref[...], 0.0)
        v = jnp.dot(z, wm3_ref[...], preferred_element_type=jnp.float32)
        out_ref[...] = jnp.exp(v + bm3_ref[...])


def _tc_readout(agg, din, b3, inst, Wm1, bm1, Wm2, bm2, Wm3, bm3):
    return pl.pallas_call(
        _tc_readout_body,
        grid=(_NBLK,),
        in_specs=[
            pl.BlockSpec((2, _RB, 32), lambda i: (0, i, 0)),
            pl.BlockSpec((_RB, 1), lambda i: (i, 0)),
            pl.BlockSpec((1, 64), lambda i: (0, 0)),
            pl.BlockSpec((1, 32), lambda i: (0, 0)),
            pl.BlockSpec((96, 256), lambda i: (0, 0)),
            pl.BlockSpec((1, 256), lambda i: (0, 0)),
            pl.BlockSpec((256, 256), lambda i: (0, 0)),
            pl.BlockSpec((1, 256), lambda i: (0, 0)),
            pl.BlockSpec((256, 1), lambda i: (0, 0)),
            pl.BlockSpec((1, 1), lambda i: (0, 0)),
        ],
        out_specs=pl.BlockSpec((1, 1), lambda i: (0, 0)),
        out_shape=jax.ShapeDtypeStruct((1, 1), jnp.float32),
        scratch_shapes=[pltpu.VMEM((1, 64), jnp.float32)],
    )(agg, din, b3, inst, Wm1, bm1, Wm2, bm2, Wm3, bm3)


def kernel(op_gid, cbo, enc, edge_index, inst_feat, emb_table, W_h, b_h,
           W1, b1, W2, b2, W3, b3, Wm1, bm1, Wm2, bm2, Wm3, bm3):
    src = edge_index[0]
    dst = edge_index[1]
    gid_pad = jnp.pad(op_gid.astype(jnp.int32), (0, NPAD - N))

    deg, emb_pad = _sc_deg_emb(src, dst, gid_pad, emb_table)
    dout = deg[0, :N, 0:1]
    din = deg[1, :N, 0:1]

    y1 = _tc_stage1(emb_pad, cbo, enc, W_h, b_h.reshape(1, 512), dout, W1)
    agg1 = _sc_scatter[128](y1, src, dst)
    y2 = _tc_mid[(256, 128)](agg1, din, dout, b1.reshape(1, 256), W2)
    agg2 = _sc_scatter[64](y2, src, dst)
    y3 = _tc_mid[(128, 64)](agg2, din, dout, b2.reshape(1, 128), W3)
    agg3 = _sc_scatter[32](y3, src, dst)

    return _tc_readout(agg3, din, b3.reshape(1, 64), inst_feat,
                       Wm1, bm1.reshape(1, 256), Wm2, bm2.reshape(1, 256),
                       Wm3, bm3.reshape(1, 1))


# speedup vs baseline: 8.5579x; 1.1045x over previous
import functools

import jax
import jax.numpy as jnp
from jax import lax
from jax.experimental import pallas as pl
from jax.experimental.pallas import tpu as pltpu
from jax.experimental.pallas import tpu_sc as plsc

N = 10000
E = 160000
NC = 2
NS = 16
EDGES_PER_TILE = E // NS
EK = 400
NCHUNKS = EDGES_PER_TILE // EK
NPAD = 10240
ROWS_PER_TILE = NPAD // NS
EMB_ROWS = NPAD // (NC * NS)

_mesh = plsc.VectorSubcoreMesh(core_axis_name="c", subcore_axis_name="s")
_sc_params = pltpu.CompilerParams(use_tc_tiling_on_sc=False)


def _zero_rows(buf, nrows, fh, dt):
    lanes = 32 if dt == jnp.bfloat16 else 16
    zv = jnp.zeros((lanes,), dt)

    def zrow(r, carry):
        for q in range(fh // lanes):
            buf[r, pl.ds(q * lanes, lanes)] = zv
        return carry
    lax.fori_loop(0, nrows, zrow, 0)


def _zero_acc_slice(buf, ek, acc_sh, rb):
    if ek >= ROWS_PER_TILE:
        pltpu.sync_copy(buf.at[pl.ds(0, ROWS_PER_TILE)],
                        acc_sh.at[pl.ds(rb, ROWS_PER_TILE)])
    else:
        pltpu.sync_copy(buf, acc_sh.at[pl.ds(rb, ek)])
        pltpu.sync_copy(buf.at[pl.ds(0, ROWS_PER_TILE - ek)],
                        acc_sh.at[pl.ds(rb + ek, ROWS_PER_TILE - ek)])


EKD = 2000
NCHUNKS_D = EDGES_PER_TILE // EKD


def _sc_deg_emb_body(src_h, dst_h, gid_h, emb_h, deg_h, embg_h,
                     id0, id1, ones_v, gidx_v, erows_v, acc_sh,
                     sem0, sem1, semg):
    c = lax.axis_index("c")
    s = lax.axis_index("s")
    rb = pl.multiple_of(s * ROWS_PER_TILE, ROWS_PER_TILE)
    _zero_rows(ones_v, ROWS_PER_TILE, 16, jnp.float32)
    _zero_acc_slice(ones_v, EKD, acc_sh, rb)
    onev = jnp.ones((16,), jnp.float32)

    def orow(r, carry):
        ones_v[r, :] = onev
        return carry
    lax.fori_loop(0, EKD, orow, 0)

    wid = s * NC + c
    gb = pl.multiple_of(wid * EMB_ROWS, 8)
    pltpu.sync_copy(gid_h.at[pl.ds(gb, EMB_ROWS)], gidx_v)
    pltpu.async_copy(emb_h.at[gidx_v], erows_v, semg)
    plsc.subcore_barrier()
    tb = s * EDGES_PER_TILE

    def deg_loop(edge_h):
        def istage(cix, idv, sem):
            eb = pl.multiple_of(tb + cix * EKD, 8)
            pltpu.async_copy(edge_h.at[pl.ds(eb, EKD)], idv, sem)

        def iwait(cix, idv, sem):
            eb = pl.multiple_of(tb + cix * EKD, 8)
            pltpu.make_async_copy(edge_h.at[pl.ds(eb, EKD)], idv, sem).wait()

        istage(0, id0, sem0)

        def pair(k, carry):
            istage(2 * k + 1, id1, sem1)
            iwait(2 * k, id0, sem0)
            pltpu.sync_copy(ones_v, acc_sh.at[id0], add=True)
            istage(2 * k + 2, id0, sem0)
            iwait(2 * k + 1, id1, sem1)
            pltpu.sync_copy(ones_v, acc_sh.at[id1], add=True)
            return carry

        lax.fori_loop(0, (NCHUNKS_D - 1) // 2, pair, 0)
        if NCHUNKS_D % 2 == 0:
            istage(NCHUNKS_D - 1, id1, sem1)
            iwait(NCHUNKS_D - 2, id0, sem0)
            pltpu.sync_copy(ones_v, acc_sh.at[id0], add=True)
            iwait(NCHUNKS_D - 1, id1, sem1)
            pltpu.sync_copy(ones_v, acc_sh.at[id1], add=True)
        else:
            iwait(NCHUNKS_D - 1, id0, sem0)
            pltpu.sync_copy(ones_v, acc_sh.at[id0], add=True)

    @pl.when(c == 0)
    def _():
        deg_loop(src_h)

    @pl.when(c == 1)
    def _():
        deg_loop(dst_h)

    pltpu.make_async_copy(emb_h.at[gidx_v], erows_v, semg).wait()
    pltpu.sync_copy(erows_v, embg_h.at[pl.ds(gb, EMB_ROWS)])

    plsc.subcore_barrier()

    @pl.when(c == 0)
    def _():
        pltpu.sync_copy(acc_sh.at[pl.ds(rb, ROWS_PER_TILE)],
                        deg_h.at[0].at[pl.ds(rb, ROWS_PER_TILE)])

    @pl.when(c == 1)
    def _():
        pltpu.sync_copy(acc_sh.at[pl.ds(rb, ROWS_PER_TILE)],
                        deg_h.at[1].at[pl.ds(rb, ROWS_PER_TILE)])


_sc_deg_emb = functools.partial(
    pl.kernel,
    out_type=(jax.ShapeDtypeStruct((2, NPAD, 16), jnp.float32),
              jax.ShapeDtypeStruct((NPAD, 32), jnp.float32)),
    mesh=_mesh,
    scratch_types=[
        pltpu.VMEM((EKD,), jnp.int32),
        pltpu.VMEM((EKD,), jnp.int32),
        pltpu.VMEM((EKD, 16), jnp.float32),
        pltpu.VMEM((EMB_ROWS,), jnp.int32),
        pltpu.VMEM((EMB_ROWS, 32), jnp.float32),
        pltpu.VMEM_SHARED((NPAD, 16), jnp.float32),
        pltpu.SemaphoreType.DMA,
        pltpu.SemaphoreType.DMA,
        pltpu.SemaphoreType.DMA,
    ],
    compiler_params=_sc_params,
)(_sc_deg_emb_body)


def _make_sc_scatter(fh, dt, ek):
    nchunks = EDGES_PER_TILE // ek

    def body(y_h, src_h, dst_h, out_h,
             is0, id0, is1, id1, r0, r1, acc_sh, sem0, sem1):
        c = lax.axis_index("c")
        s = lax.axis_index("s")
        rb = pl.multiple_of(s * ROWS_PER_TILE, ROWS_PER_TILE)
        _zero_rows(r0, min(ek, ROWS_PER_TILE), fh, dt)
        _zero_acc_slice(r0, ek, acc_sh, rb)
        plsc.subcore_barrier()
        tb = s * EDGES_PER_TILE

        def stage(cix, isv, idv):
            eb = pl.multiple_of(tb + cix * ek, 8)
            pltpu.sync_copy(src_h.at[pl.ds(eb, ek)], isv)
            pltpu.sync_copy(dst_h.at[pl.ds(eb, ek)], idv)

        def gather(isv, rv, sem):
            @pl.when(c == 0)
            def _():
                pltpu.async_copy(y_h.at[0].at[isv], rv, sem)

            @pl.when(c == 1)
            def _():
                pltpu.async_copy(y_h.at[1].at[isv], rv, sem)

        def gwait(isv, rv, sem):
            @pl.when(c == 0)
            def _():
                pltpu.make_async_copy(y_h.at[0].at[isv], rv, sem).wait()

            @pl.when(c == 1)
            def _():
                pltpu.make_async_copy(y_h.at[1].at[isv], rv, sem).wait()

        stage(0, is0, id0)
        gather(is0, r0, sem0)

        def pair(k, carry):
            stage(2 * k + 1, is1, id1)
            gather(is1, r1, sem1)
            gwait(is0, r0, sem0)
            pltpu.sync_copy(r0, acc_sh.at[id0], add=True)
            stage(2 * k + 2, is0, id0)
            gather(is0, r0, sem0)
            gwait(is1, r1, sem1)
            pltpu.sync_copy(r1, acc_sh.at[id1], add=True)
            return carry

        lax.fori_loop(0, (nchunks - 1) // 2, pair, 0)
        if nchunks % 2 == 0:
            stage(nchunks - 1, is1, id1)
            gather(is1, r1, sem1)
            gwait(is0, r0, sem0)
            pltpu.sync_copy(r0, acc_sh.at[id0], add=True)
            gwait(is1, r1, sem1)
            pltpu.sync_copy(r1, acc_sh.at[id1], add=True)
        else:
            gwait(is0, r0, sem0)
            pltpu.sync_copy(r0, acc_sh.at[id0], add=True)
        plsc.subcore_barrier()

        @pl.when(c == 0)
        def _():
            pltpu.sync_copy(acc_sh.at[pl.ds(rb, ROWS_PER_TILE)],
                            out_h.at[0].at[pl.ds(rb, ROWS_PER_TILE)])

        @pl.when(c == 1)
        def _():
            pltpu.sync_copy(acc_sh.at[pl.ds(rb, ROWS_PER_TILE)],
                            out_h.at[1].at[pl.ds(rb, ROWS_PER_TILE)])

    return pl.kernel(
        body,
        out_type=jax.ShapeDtypeStruct((2, NPAD, fh), dt),
        mesh=_mesh,
        scratch_types=[
            pltpu.VMEM((ek,), jnp.int32),
            pltpu.VMEM((ek,), jnp.int32),
            pltpu.VMEM((ek,), jnp.int32),
            pltpu.VMEM((ek,), jnp.int32),
            pltpu.VMEM((ek, fh), dt),
            pltpu.VMEM((ek, fh), dt),
            pltpu.VMEM_SHARED((NPAD, fh), dt),
            pltpu.SemaphoreType.DMA,
            pltpu.SemaphoreType.DMA,
        ],
        compiler_params=_sc_params,
    )


_sc_scatter = {128: _make_sc_scatter(128, jnp.bfloat16, 400),
               64: _make_sc_scatter(64, jnp.bfloat16, 1000),
               32: _make_sc_scatter(32, jnp.bfloat16, 2000)}


_RB = 2000
_NBLK = N // _RB


def _norm(d):
    return lax.rsqrt(jnp.where(d > 0.0, d, 1.0))


def _tc_stage1_body(emb_ref, cbo_ref, enc_ref, wh_ref, bh_ref, dout_ref,
                    w1_ref, out_ref):
    t = jnp.dot(emb_ref[...], wh_ref[0:32, :],
                preferred_element_type=jnp.float32)
    t += jnp.dot(cbo_ref[...], wh_ref[32:128, :],
                 preferred_element_type=jnp.float32)
    t += jnp.dot(enc_ref[...], wh_ref[128:256, :],
                 preferred_element_type=jnp.float32)
    t = jnp.maximum(t + bh_ref[...], 0.0)
    x = t * _norm(dout_ref[...])
    y = jnp.dot(x, w1_ref[...], preferred_element_type=jnp.float32)
    yb = y.astype(jnp.bfloat16)
    out_ref[0] = yb[:, :128]
    out_ref[1] = yb[:, 128:]


def _tc_stage1(emb_g, cbo, enc, W_h, b_h, dout, W1):
    return pl.pallas_call(
        _tc_stage1_body,
        grid=(_NBLK,),
        in_specs=[
            pl.BlockSpec((_RB, 32), lambda i: (i, 0)),
            pl.BlockSpec((_RB, 96), lambda i: (i, 0)),
            pl.BlockSpec((_RB, 128), lambda i: (i, 0)),
            pl.BlockSpec((256, 512), lambda i: (0, 0)),
            pl.BlockSpec((1, 512), lambda i: (0, 0)),
            pl.BlockSpec((_RB, 1), lambda i: (i, 0)),
            pl.BlockSpec((512, 256), lambda i: (0, 0)),
        ],
        out_specs=pl.BlockSpec((2, _RB, 128), lambda i: (0, i, 0)),
        out_shape=jax.ShapeDtypeStruct((2, N, 128), jnp.bfloat16),
    )(emb_g, cbo, enc, W_h, b_h, dout, W1)


def _make_tc_mid(fin, fout, in_dt):
    fi2, fo2 = fin // 2, fout // 2

    def body(agg_ref, din_ref, dout_ref, b_ref, w_ref, out_ref):
        nd = _norm(din_ref[...])
        ns = _norm(dout_ref[...])
        a0 = agg_ref[0].astype(jnp.float32)
        a1 = agg_ref[1].astype(jnp.float32)
        x0 = jnp.maximum(a0 * nd + b_ref[:, :fi2], 0.0) * ns
        x1 = jnp.maximum(a1 * nd + b_ref[:, fi2:], 0.0) * ns
        y = jnp.dot(x0, w_ref[:fi2, :], preferred_element_type=jnp.float32)
        y += jnp.dot(x1, w_ref[fi2:, :], preferred_element_type=jnp.float32)
        yb = y.astype(jnp.bfloat16)
        out_ref[0] = yb[:, :fo2]
        out_ref[1] = yb[:, fo2:]

    def run(agg, din, dout, b, w):
        return pl.pallas_call(
            body,
            grid=(_NBLK,),
            in_specs=[
                pl.BlockSpec((2, _RB, fi2), lambda i: (0, i, 0)),
                pl.BlockSpec((_RB, 1), lambda i: (i, 0)),
                pl.BlockSpec((_RB, 1), lambda i: (i, 0)),
                pl.BlockSpec((1, fin), lambda i: (0, 0)),
                pl.BlockSpec((fin, fout), lambda i: (0, 0)),
            ],
            out_specs=pl.BlockSpec((2, _RB, fo2), lambda i: (0, i, 0)),
            out_shape=jax.ShapeDtypeStruct((2, N, fo2), jnp.bfloat16),
        )(agg, din, dout, b, w)

    return run


_tc_mid = {(256, 128): _make_tc_mid(256, 128, jnp.bfloat16),
           (128, 64): _make_tc_mid(128, 64, jnp.float32)}


def _tc_readout_body(agg_ref, din_ref, b3_ref, inst_ref, wm1_ref, bm1_ref,
                     wm2_ref, bm2_ref, wm3_ref, bm3_ref, out_ref, acc_ref):
    i = pl.program_id(0)

    @pl.when(i == 0)
    def _():
        acc_ref[...] = jnp.zeros_like(acc_ref)

    nd = _norm(din_ref[...])
    x0 = jnp.maximum(agg_ref[0].astype(jnp.float32) * nd + b3_ref[:, :32], 0.0)
    x1 = jnp.maximum(agg_ref[1].astype(jnp.float32) * nd + b3_ref[:, 32:], 0.0)
    acc_ref[:, :32] += jnp.sum(x0, axis=0, keepdims=True)
    acc_ref[:, 32:] += jnp.sum(x1, axis=0, keepdims=True)

    @pl.when(i == _NBLK - 1)
    def _():
        hg = acc_ref[...] * (1.0 / N)
        z = jnp.dot(hg, wm1_ref[:64, :], preferred_element_type=jnp.float32)
        z += jnp.dot(inst_ref[...], wm1_ref[64:, :],
                     preferred_element_type=jnp.float32)
        z = jnp.maximum(z + bm1_ref[...], 0.0)
        z = jnp.maximum(jnp.dot(z, wm2_ref[...],
                                preferred_element_type=jnp.float32)
                        + bm2_ref[...], 0.0)
        v = jnp.dot(z, wm3_ref[...], preferred_element_type=jnp.float32)
        out_ref[...] = jnp.exp(v + bm3_ref[...])


def _tc_readout(agg, din, b3, inst, Wm1, bm1, Wm2, bm2, Wm3, bm3):
    return pl.pallas_call(
        _tc_readout_body,
        grid=(_NBLK,),
        in_specs=[
            pl.BlockSpec((2, _RB, 32), lambda i: (0, i, 0)),
            pl.BlockSpec((_RB, 1), lambda i: (i, 0)),
            pl.BlockSpec((1, 64), lambda i: (0, 0)),
            pl.BlockSpec((1, 32), lambda i: (0, 0)),
            pl.BlockSpec((96, 256), lambda i: (0, 0)),
            pl.BlockSpec((1, 256), lambda i: (0, 0)),
            pl.BlockSpec((256, 256), lambda i: (0, 0)),
            pl.BlockSpec((1, 256), lambda i: (0, 0)),
            pl.BlockSpec((256, 1), lambda i: (0, 0)),
            pl.BlockSpec((1, 1), lambda i: (0, 0)),
        ],
        out_specs=pl.BlockSpec((1, 1), lambda i: (0, 0)),
        out_shape=jax.ShapeDtypeStruct((1, 1), jnp.float32),
        scratch_shapes=[pltpu.VMEM((1, 64), jnp.float32)],
    )(agg, din, b3, inst, Wm1, bm1, Wm2, bm2, Wm3, bm3)


def kernel(op_gid, cbo, enc, edge_index, inst_feat, emb_table, W_h, b_h,
           W1, b1, W2, b2, W3, b3, Wm1, bm1, Wm2, bm2, Wm3, bm3):
    src = edge_index[0]
    dst = edge_index[1]
    gid_pad = jnp.pad(op_gid.astype(jnp.int32), (0, NPAD - N))

    deg, emb_pad = _sc_deg_emb(src, dst, gid_pad, emb_table)
    dout = deg[0, :N, 0:1]
    din = deg[1, :N, 0:1]

    y1 = _tc_stage1(emb_pad, cbo, enc, W_h, b_h.reshape(1, 512), dout, W1)
    agg1 = _sc_scatter[128](y1, src, dst)
    y2 = _tc_mid[(256, 128)](agg1, din, dout, b1.reshape(1, 256), W2)
    agg2 = _sc_scatter[64](y2, src, dst)
    y3 = _tc_mid[(128, 64)](agg2, din, dout, b2.reshape(1, 128), W3)
    agg3 = _sc_scatter[32](y3, src, dst)

    return _tc_readout(agg3, din, b3.reshape(1, 64), inst_feat,
                       Wm1, bm1.reshape(1, 256), Wm2, bm2.reshape(1, 256),
                       Wm3, bm3.reshape(1, 1))


# speedup vs baseline: 8.5655x; 1.0009x over previous
import functools

import jax
import jax.numpy as jnp
from jax import lax
from jax.experimental import pallas as pl
from jax.experimental.pallas import tpu as pltpu
from jax.experimental.pallas import tpu_sc as plsc

N = 10000
E = 160000
NC = 2
NS = 16
EDGES_PER_TILE = E // NS
EK = 400
NCHUNKS = EDGES_PER_TILE // EK
NPAD = 10240
ROWS_PER_TILE = NPAD // NS
EMB_ROWS = NPAD // (NC * NS)

_mesh = plsc.VectorSubcoreMesh(core_axis_name="c", subcore_axis_name="s")
_sc_params = pltpu.CompilerParams(use_tc_tiling_on_sc=False)


def _zero_rows(buf, nrows, fh, dt):
    lanes = 32 if dt == jnp.bfloat16 else 16
    zv = jnp.zeros((lanes,), dt)

    def zrow(r, carry):
        for q in range(fh // lanes):
            buf[r, pl.ds(q * lanes, lanes)] = zv
        return carry
    lax.fori_loop(0, nrows, zrow, 0)


def _zero_acc_slice(buf, ek, acc_sh, rb):
    if ek >= ROWS_PER_TILE:
        pltpu.sync_copy(buf.at[pl.ds(0, ROWS_PER_TILE)],
                        acc_sh.at[pl.ds(rb, ROWS_PER_TILE)])
    else:
        pltpu.sync_copy(buf, acc_sh.at[pl.ds(rb, ek)])
        pltpu.sync_copy(buf.at[pl.ds(0, ROWS_PER_TILE - ek)],
                        acc_sh.at[pl.ds(rb + ek, ROWS_PER_TILE - ek)])


EKD = 2000
NCHUNKS_D = EDGES_PER_TILE // EKD


def _sc_deg_emb_body(src_h, dst_h, gid_h, emb_h, deg_h, embg_h,
                     id0, id1, ones_v, gidx_v, erows_v, acc_sh,
                     sem0, sem1, semg):
    c = lax.axis_index("c")
    s = lax.axis_index("s")
    rb = pl.multiple_of(s * ROWS_PER_TILE, ROWS_PER_TILE)
    _zero_rows(ones_v, ROWS_PER_TILE, 16, jnp.float32)
    _zero_acc_slice(ones_v, EKD, acc_sh, rb)
    onev = jnp.ones((16,), jnp.float32)

    def orow(r, carry):
        ones_v[r, :] = onev
        return carry
    lax.fori_loop(0, EKD, orow, 0)

    wid = s * NC + c
    gb = pl.multiple_of(wid * EMB_ROWS, 8)
    pltpu.sync_copy(gid_h.at[pl.ds(gb, EMB_ROWS)], gidx_v)
    pltpu.async_copy(emb_h.at[gidx_v], erows_v, semg)
    plsc.subcore_barrier()
    tb = s * EDGES_PER_TILE

    def deg_loop(edge_h):
        def istage(cix, idv, sem):
            eb = pl.multiple_of(tb + cix * EKD, 8)
            pltpu.async_copy(edge_h.at[pl.ds(eb, EKD)], idv, sem)

        def iwait(cix, idv, sem):
            eb = pl.multiple_of(tb + cix * EKD, 8)
            pltpu.make_async_copy(edge_h.at[pl.ds(eb, EKD)], idv, sem).wait()

        istage(0, id0, sem0)

        def pair(k, carry):
            istage(2 * k + 1, id1, sem1)
            iwait(2 * k, id0, sem0)
            pltpu.sync_copy(ones_v, acc_sh.at[id0], add=True)
            istage(2 * k + 2, id0, sem0)
            iwait(2 * k + 1, id1, sem1)
            pltpu.sync_copy(ones_v, acc_sh.at[id1], add=True)
            return carry

        lax.fori_loop(0, (NCHUNKS_D - 1) // 2, pair, 0)
        if NCHUNKS_D % 2 == 0:
            istage(NCHUNKS_D - 1, id1, sem1)
            iwait(NCHUNKS_D - 2, id0, sem0)
            pltpu.sync_copy(ones_v, acc_sh.at[id0], add=True)
            iwait(NCHUNKS_D - 1, id1, sem1)
            pltpu.sync_copy(ones_v, acc_sh.at[id1], add=True)
        else:
            iwait(NCHUNKS_D - 1, id0, sem0)
            pltpu.sync_copy(ones_v, acc_sh.at[id0], add=True)

    @pl.when(c == 0)
    def _():
        deg_loop(src_h)

    @pl.when(c == 1)
    def _():
        deg_loop(dst_h)

    pltpu.make_async_copy(emb_h.at[gidx_v], erows_v, semg).wait()
    pltpu.sync_copy(erows_v, embg_h.at[pl.ds(gb, EMB_ROWS)])

    plsc.subcore_barrier()

    @pl.when(c == 0)
    def _():
        pltpu.sync_copy(acc_sh.at[pl.ds(rb, ROWS_PER_TILE), pl.ds(0, 8)],
                        deg_h.at[0].at[pl.ds(rb, ROWS_PER_TILE)])

    @pl.when(c == 1)
    def _():
        pltpu.sync_copy(acc_sh.at[pl.ds(rb, ROWS_PER_TILE), pl.ds(0, 8)],
                        deg_h.at[1].at[pl.ds(rb, ROWS_PER_TILE)])


_sc_deg_emb = functools.partial(
    pl.kernel,
    out_type=(jax.ShapeDtypeStruct((2, NPAD, 8), jnp.float32),
              jax.ShapeDtypeStruct((NPAD, 32), jnp.float32)),
    mesh=_mesh,
    scratch_types=[
        pltpu.VMEM((EKD,), jnp.int32),
        pltpu.VMEM((EKD,), jnp.int32),
        pltpu.VMEM((EKD, 16), jnp.float32),
        pltpu.VMEM((EMB_ROWS,), jnp.int32),
        pltpu.VMEM((EMB_ROWS, 32), jnp.float32),
        pltpu.VMEM_SHARED((NPAD, 16), jnp.float32),
        pltpu.SemaphoreType.DMA,
        pltpu.SemaphoreType.DMA,
        pltpu.SemaphoreType.DMA,
    ],
    compiler_params=_sc_params,
)(_sc_deg_emb_body)


def _make_sc_scatter(fh, dt, ek):
    nchunks = EDGES_PER_TILE // ek

    def body(y_h, src_h, dst_h, out_h,
             is0, id0, is1, id1, r0, r1, acc_sh, sem0, sem1):
        c = lax.axis_index("c")
        s = lax.axis_index("s")
        rb = pl.multiple_of(s * ROWS_PER_TILE, ROWS_PER_TILE)
        _zero_rows(r0, min(ek, ROWS_PER_TILE), fh, dt)
        _zero_acc_slice(r0, ek, acc_sh, rb)
        plsc.subcore_barrier()
        tb = s * EDGES_PER_TILE

        def stage(cix, isv, idv):
            eb = pl.multiple_of(tb + cix * ek, 8)
            pltpu.sync_copy(src_h.at[pl.ds(eb, ek)], isv)
            pltpu.sync_copy(dst_h.at[pl.ds(eb, ek)], idv)

        def gather(isv, rv, sem):
            @pl.when(c == 0)
            def _():
                pltpu.async_copy(y_h.at[0].at[isv], rv, sem)

            @pl.when(c == 1)
            def _():
                pltpu.async_copy(y_h.at[1].at[isv], rv, sem)

        def gwait(isv, rv, sem):
            @pl.when(c == 0)
            def _():
                pltpu.make_async_copy(y_h.at[0].at[isv], rv, sem).wait()

            @pl.when(c == 1)
            def _():
                pltpu.make_async_copy(y_h.at[1].at[isv], rv, sem).wait()

        stage(0, is0, id0)
        gather(is0, r0, sem0)

        def pair(k, carry):
            stage(2 * k + 1, is1, id1)
            gather(is1, r1, sem1)
            gwait(is0, r0, sem0)
            pltpu.sync_copy(r0, acc_sh.at[id0], add=True)
            stage(2 * k + 2, is0, id0)
            gather(is0, r0, sem0)
            gwait(is1, r1, sem1)
            pltpu.sync_copy(r1, acc_sh.at[id1], add=True)
            return carry

        lax.fori_loop(0, (nchunks - 1) // 2, pair, 0)
        if nchunks % 2 == 0:
            stage(nchunks - 1, is1, id1)
            gather(is1, r1, sem1)
            gwait(is0, r0, sem0)
            pltpu.sync_copy(r0, acc_sh.at[id0], add=True)
            gwait(is1, r1, sem1)
            pltpu.sync_copy(r1, acc_sh.at[id1], add=True)
        else:
            gwait(is0, r0, sem0)
            pltpu.sync_copy(r0, acc_sh.at[id0], add=True)
        plsc.subcore_barrier()

        @pl.when(c == 0)
        def _():
            pltpu.sync_copy(acc_sh.at[pl.ds(rb, ROWS_PER_TILE)],
                            out_h.at[0].at[pl.ds(rb, ROWS_PER_TILE)])

        @pl.when(c == 1)
        def _():
            pltpu.sync_copy(acc_sh.at[pl.ds(rb, ROWS_PER_TILE)],
                            out_h.at[1].at[pl.ds(rb, ROWS_PER_TILE)])

    return pl.kernel(
        body,
        out_type=jax.ShapeDtypeStruct((2, NPAD, fh), dt),
        mesh=_mesh,
        scratch_types=[
            pltpu.VMEM((ek,), jnp.int32),
            pltpu.VMEM((ek,), jnp.int32),
            pltpu.VMEM((ek,), jnp.int32),
            pltpu.VMEM((ek,), jnp.int32),
            pltpu.VMEM((ek, fh), dt),
            pltpu.VMEM((ek, fh), dt),
            pltpu.VMEM_SHARED((NPAD, fh), dt),
            pltpu.SemaphoreType.DMA,
            pltpu.SemaphoreType.DMA,
        ],
        compiler_params=_sc_params,
    )


_sc_scatter = {128: _make_sc_scatter(128, jnp.bfloat16, 400),
               64: _make_sc_scatter(64, jnp.bfloat16, 1000),
               32: _make_sc_scatter(32, jnp.bfloat16, 2000)}


_RB = 2000
_NBLK = N // _RB


def _norm(d):
    return lax.rsqrt(jnp.where(d > 0.0, d, 1.0))


def _tc_stage1_body(emb_ref, cbo_ref, enc_ref, wh_ref, bh_ref, deg_ref,
                    w1_ref, out_ref):
    t = jnp.dot(emb_ref[...], wh_ref[0:32, :],
                preferred_element_type=jnp.float32)
    t += jnp.dot(cbo_ref[...], wh_ref[32:128, :],
                 preferred_element_type=jnp.float32)
    t += jnp.dot(enc_ref[...], wh_ref[128:256, :],
                 preferred_element_type=jnp.float32)
    t = jnp.maximum(t + bh_ref[...], 0.0)
    x = t * _norm(deg_ref[0, :, 0:1])
    y = jnp.dot(x, w1_ref[...], preferred_element_type=jnp.float32)
    yb = y.astype(jnp.bfloat16)
    out_ref[0] = yb[:, :128]
    out_ref[1] = yb[:, 128:]


def _tc_stage1(emb_g, cbo, enc, W_h, b_h, deg, W1):
    return pl.pallas_call(
        _tc_stage1_body,
        grid=(_NBLK,),
        in_specs=[
            pl.BlockSpec((_RB, 32), lambda i: (i, 0)),
            pl.BlockSpec((_RB, 96), lambda i: (i, 0)),
            pl.BlockSpec((_RB, 128), lambda i: (i, 0)),
            pl.BlockSpec((256, 512), lambda i: (0, 0)),
            pl.BlockSpec((1, 512), lambda i: (0, 0)),
            pl.BlockSpec((2, _RB, 8), lambda i: (0, i, 0)),
            pl.BlockSpec((512, 256), lambda i: (0, 0)),
        ],
        out_specs=pl.BlockSpec((2, _RB, 128), lambda i: (0, i, 0)),
        out_shape=jax.ShapeDtypeStruct((2, N, 128), jnp.bfloat16),
    )(emb_g, cbo, enc, W_h, b_h, deg, W1)


def _make_tc_mid(fin, fout, in_dt):
    fi2, fo2 = fin // 2, fout // 2

    def body(agg_ref, deg_ref, b_ref, w_ref, out_ref):
        nd = _norm(deg_ref[1, :, 0:1])
        ns = _norm(deg_ref[0, :, 0:1])
        a0 = agg_ref[0].astype(jnp.float32)
        a1 = agg_ref[1].astype(jnp.float32)
        x0 = jnp.maximum(a0 * nd + b_ref[:, :fi2], 0.0) * ns
        x1 = jnp.maximum(a1 * nd + b_ref[:, fi2:], 0.0) * ns
        y = jnp.dot(x0, w_ref[:fi2, :], preferred_element_type=jnp.float32)
        y += jnp.dot(x1, w_ref[fi2:, :], preferred_element_type=jnp.float32)
        yb = y.astype(jnp.bfloat16)
        out_ref[0] = yb[:, :fo2]
        out_ref[1] = yb[:, fo2:]

    def run(agg, deg, b, w):
        return pl.pallas_call(
            body,
            grid=(_NBLK,),
            in_specs=[
                pl.BlockSpec((2, _RB, fi2), lambda i: (0, i, 0)),
                pl.BlockSpec((2, _RB, 8), lambda i: (0, i, 0)),
                pl.BlockSpec((1, fin), lambda i: (0, 0)),
                pl.BlockSpec((fin, fout), lambda i: (0, 0)),
            ],
            out_specs=pl.BlockSpec((2, _RB, fo2), lambda i: (0, i, 0)),
            out_shape=jax.ShapeDtypeStruct((2, N, fo2), jnp.bfloat16),
        )(agg, deg, b, w)

    return run


_tc_mid = {(256, 128): _make_tc_mid(256, 128, jnp.bfloat16),
           (128, 64): _make_tc_mid(128, 64, jnp.float32)}


def _tc_readout_body(agg_ref, deg_ref, b3_ref, inst_ref, wm1_ref, bm1_ref,
                     wm2_ref, bm2_ref, wm3_ref, bm3_ref, out_ref, acc_ref):
    i = pl.program_id(0)

    @pl.when(i == 0)
    def _():
        acc_ref[...] = jnp.zeros_like(acc_ref)

    nd = _norm(deg_ref[1, :, 0:1])
    x0 = jnp.maximum(agg_ref[0].astype(jnp.float32) * nd + b3_ref[:, :32], 0.0)
    x1 = jnp.maximum(agg_ref[1].astype(jnp.float32) * nd + b3_ref[:, 32:], 0.0)
    acc_ref[:, :32] += jnp.sum(x0, axis=0, keepdims=True)
    acc_ref[:, 32:] += jnp.sum(x1, axis=0, keepdims=True)

    @pl.when(i == _NBLK - 1)
    def _():
        hg = acc_ref[...] * (1.0 / N)
        z = jnp.dot(hg, wm1_ref[:64, :], preferred_element_type=jnp.float32)
        z += jnp.dot(inst_ref[...], wm1_ref[64:, :],
                     preferred_element_type=jnp.float32)
        z = jnp.maximum(z + bm1_ref[...], 0.0)
        z = jnp.maximum(jnp.dot(z, wm2_ref[...],
                                preferred_element_type=jnp.float32)
                        + bm2_ref[...], 0.0)
        v = jnp.dot(z, wm3_ref[...], preferred_element_type=jnp.float32)
        out_ref[...] = jnp.exp(v + bm3_ref[...])


def _tc_readout(agg, deg, b3, inst, Wm1, bm1, Wm2, bm2, Wm3, bm3):
    return pl.pallas_call(
        _tc_readout_body,
        grid=(_NBLK,),
        in_specs=[
            pl.BlockSpec((2, _RB, 32), lambda i: (0, i, 0)),
            pl.BlockSpec((2, _RB, 8), lambda i: (0, i, 0)),
            pl.BlockSpec((1, 64), lambda i: (0, 0)),
            pl.BlockSpec((1, 32), lambda i: (0, 0)),
            pl.BlockSpec((96, 256), lambda i: (0, 0)),
            pl.BlockSpec((1, 256), lambda i: (0, 0)),
            pl.BlockSpec((256, 256), lambda i: (0, 0)),
            pl.BlockSpec((1, 256), lambda i: (0, 0)),
            pl.BlockSpec((256, 1), lambda i: (0, 0)),
            pl.BlockSpec((1, 1), lambda i: (0, 0)),
        ],
        out_specs=pl.BlockSpec((1, 1), lambda i: (0, 0)),
        out_shape=jax.ShapeDtypeStruct((1, 1), jnp.float32),
        scratch_shapes=[pltpu.VMEM((1, 64), jnp.float32)],
    )(agg, deg, b3, inst, Wm1, bm1, Wm2, bm2, Wm3, bm3)


def kernel(op_gid, cbo, enc, edge_index, inst_feat, emb_table, W_h, b_h,
           W1, b1, W2, b2, W3, b3, Wm1, bm1, Wm2, bm2, Wm3, bm3):
    src = edge_index[0]
    dst = edge_index[1]
    gid_pad = jnp.pad(op_gid.astype(jnp.int32), (0, NPAD - N))

    deg, emb_pad = _sc_deg_emb(src, dst, gid_pad, emb_table)
    y1 = _tc_stage1(emb_pad, cbo, enc, W_h, b_h.reshape(1, 512), deg, W1)
    agg1 = _sc_scatter[128](y1, src, dst)
    y2 = _tc_mid[(256, 128)](agg1, deg, b1.reshape(1, 256), W2)
    agg2 = _sc_scatter[64](y2, src, dst)
    y3 = _tc_mid[(128, 64)](agg2, deg, b2.reshape(1, 128), W3)
    agg3 = _sc_scatter[32](y3, src, dst)

    return _tc_readout(agg3, deg, b3.reshape(1, 64), inst_feat,
                       Wm1, bm1.reshape(1, 256), Wm2, bm2.reshape(1, 256),
                       Wm3, bm3.reshape(1, 1))
